# baseline (speedup 1.0000x reference)
import jax
import jax.numpy as jnp
from jax import lax
from jax.experimental import pallas as pl
from jax.experimental.pallas import tpu as pltpu
from jax.experimental.pallas import tpu_sc as plsc

N = 10000
E = 160000
D = 256
NC, NS, L = 2, 16, 16
NT = NC * NS
RT = 320
CH = 32
RW = 128
CPG = RW // CH
CPR = 12
NIR = 6
EP = 160128
NRW = EP // RW
FB = 8
ACCW = 64
TRASH = ACCW * D
PROP = 5

_mesh = plsc.VectorSubcoreMesh(
    core_axis_name="c", subcore_axis_name="s", num_cores=NC, num_subcores=NS)


def _rsqrt_vec(d):
    bits = lax.bitcast_convert_type(d, jnp.int32)
    y = lax.bitcast_convert_type(jnp.int32(0x5F3759DF) - (bits >> 1),
                                 jnp.float32)
    for _ in range(3):
        y = y * (1.5 - 0.5 * d * y * y)
    return y


def _span(offs_hbm, obuf, t):
    pltpu.sync_copy(offs_hbm, obuf)
    ov = obuf[pl.ds(t, L)]
    start = ov[0]
    end = ov[1]
    ra = lax.div(start, RW)
    rb = lax.div(end + (RW - 1), RW)
    lo = t * RT
    hi = jnp.minimum(lo + RT, N)
    return ra, rb, lo, hi


def _step_body(srows_hbm, drows_hbm, offs_hbm, w_hbm, x_hbm, zin_hbm,
               zout_hbm, yout_hbm, acc, nwin, gbuf, sidx, didx, obuf, xb, yb,
               zb, wf, gsems, isems):
    c = lax.axis_index("c")
    s = lax.axis_index("s")
    t = c * NS + s
    ra, rb, lo, hi = _span(offs_hbm, obuf, t)
    zeros16 = jnp.zeros((L,), jnp.float32)
    ca = ra * CPG
    cb = rb * CPG

    nfill = jnp.full((L,), N, jnp.int32)

    @pl.loop(0, ACCW)
    def _(r):
        nwin[pl.ds(r * L, L)] = nfill

    def flush(rcur, wrank, rsums):
        inr = jnp.logical_and(rcur >= lo, rcur < hi)
        slot = lax.rem(wrank, ACCW)
        base = jnp.where(inr, slot * D, TRASH)
        nwin[pl.ds(slot * L, L)] = jnp.full((L,), rcur, jnp.int32)
        for q in range(D // L):
            acc[pl.ds(base + q * L, L)] = rsums[q]

    def sweep(fdone, rrank, swp):
        nblk = (swp - fdone) >> 3

        def block(b, rk):
            n0 = fdone + b * FB
            pltpu.sync_copy(x_hbm.at[pl.ds(n0 * D, FB * D)], xb)
            woff = pl.multiple_of((n0 - lo) * L, FB * L)
            pltpu.sync_copy(w_hbm.at[t].at[pl.ds(woff, FB * L)], wf)

            def node(r, rk2):
                slot = lax.rem(rk2, ACCW)
                nid = nwin[pl.ds(slot * L, L)][0]
                match = nid == n0 + r
                dv = wf[pl.ds(r * L, L)][0]
                for q in range(D // L):
                    o = r * D + q * L
                    a = acc[pl.ds(slot * D + q * L, L)]
                    y = (0.5 * dv * jnp.where(match, a, zeros16)
                         + 0.5 * xb[pl.ds(o, L)])
                    yb[pl.ds(o, L)] = y
                    zb[pl.ds(o, L)] = dv * y
                return rk2 + match.astype(jnp.int32)

            rk = pl.loop(0, FB, init_carry=rk)(node)
            pltpu.sync_copy(yb, yout_hbm.at[pl.ds(n0 * D, FB * D)])
            pltpu.sync_copy(zb, zout_hbm.at[pl.ds(n0 * D, FB * D)])
            return rk

        rrank = pl.loop(0, nblk, init_carry=rrank)(block)
        return fdone + nblk * FB, rrank

    def row_group(gslot, ioff, j, rc):
        rcur = rc[0]
        wrank = rc[1]
        rsums = rc[2:]
        dvec = didx[pl.ds(ioff + j * L, L)]
        for l in range(L):
            r = j * L + l
            dstv = dvec[l]
            changed = dstv != rcur

            @pl.when(changed)
            def _():
                flush(rcur, wrank, rsums)

            inr = jnp.logical_and(rcur >= lo, rcur < hi)
            wrank = jnp.where(jnp.logical_and(changed, inr), wrank + 1, wrank)
            rsums = tuple(
                jnp.where(changed, gbuf[gslot, r, pl.ds(q * L, L)],
                          rsums[q] + gbuf[gslot, r, pl.ds(q * L, L)])
                for q in range(D // L))
            rcur = jnp.where(changed, dstv, rcur)
        return (rcur, wrank) + rsums

    def stage_row(r, sync):
        rr = jnp.minimum(r, NRW - 1)
        off = lax.rem(rr, NIR) * RW
        if sync:
            pltpu.sync_copy(srows_hbm.at[rr], sidx.at[pl.ds(off, RW)])
            pltpu.sync_copy(drows_hbm.at[rr], didx.at[pl.ds(off, RW)])
        else:
            sem = isems.at[lax.rem(rr, NIR)]
            pltpu.async_copy(srows_hbm.at[rr], sidx.at[pl.ds(off, RW)], sem)
            pltpu.async_copy(drows_hbm.at[rr], didx.at[pl.ds(off, RW)], sem)

    def wait_row(r):
        rr = jnp.minimum(r, NRW - 1)
        off = lax.rem(rr, NIR) * RW
        sem = isems.at[lax.rem(rr, NIR)]
        pltpu.make_async_copy(srows_hbm.at[rr], sidx.at[pl.ds(off, RW)],
                              sem).wait()
        pltpu.make_async_copy(drows_hbm.at[rr], didx.at[pl.ds(off, RW)],
                              sem).wait()

    def issue(ch):
        irow = lax.div(ch, CPG)
        ioff = (lax.rem(irow, NIR) * RW + lax.rem(ch, CPG) * CH)
        pltpu.async_copy(zin_hbm.at[sidx.at[pl.ds(ioff, CH)]],
                         gbuf.at[lax.rem(ch, CPR)], gsems.at[lax.rem(ch, CPR)])

    for i in range(3):
        stage_row(ra + i, True)
    for i in range(3, 5):
        stage_row(ra + i, False)
    for i in range(CPR):
        @pl.when(ca + i < cb)
        def _(i=i):
            issue(ca + i)

    def chunk_body(ch, carry):
        rcur, wrank, fdone, rrank = carry[:4]
        rsums = carry[4:]
        irow = lax.div(ch, CPG)
        gslot = lax.rem(ch, CPR)
        ioff = lax.rem(irow, NIR) * RW + lax.rem(ch, CPG) * CH

        @pl.when(jnp.logical_or(lax.rem(ch, CPG) == 0, ch == ca))
        def _():
            wait_row(irow + 3)
            stage_row(irow + 5, False)

        pltpu.make_async_copy(zin_hbm.at[sidx.at[pl.ds(ioff, CH)]],
                              gbuf.at[gslot], gsems.at[gslot]).wait()

        rc = (rcur, wrank) + rsums
        rc = pl.loop(0, CH // L, init_carry=rc)(
            lambda j, c: row_group(gslot, ioff, j, c))

        @pl.when(ch + CPR < cb)
        def _():
            issue(ch + CPR)

        lim = didx[pl.ds(ioff + CH - L, L)][L - 1]
        swp = jnp.maximum(lo, jnp.minimum(lim, hi))
        fdone, rrank = sweep(fdone, rrank, swp)
        return (rc[0], rc[1], fdone, rrank) + rc[2:]

    init = ((jnp.int32(N), jnp.int32(0), lo, jnp.int32(0))
            + tuple(zeros16 for _ in range(D // L)))
    carry = pl.loop(ca, cb, init_carry=init)(chunk_body)
    flush(carry[0], carry[1], carry[4:])
    sweep(carry[2], carry[3], hi)


_step = pl.kernel(
    _step_body,
    out_type=(jax.ShapeDtypeStruct((N * D,), jnp.float32),
              jax.ShapeDtypeStruct((N * D,), jnp.float32)),
    mesh=_mesh,
    scratch_types=[
        pltpu.VMEM((ACCW * D + D,), jnp.float32),
        pltpu.VMEM((ACCW * L,), jnp.int32),
        pltpu.VMEM((CPR, CH, D), jnp.float32),
        pltpu.VMEM((NIR * RW,), jnp.int32),
        pltpu.VMEM((NIR * RW,), jnp.int32),
        pltpu.VMEM((NT + L,), jnp.int32),
        pltpu.VMEM((FB * D,), jnp.float32),
        pltpu.VMEM((FB * D,), jnp.float32),
        pltpu.VMEM((FB * D,), jnp.float32),
        pltpu.VMEM((FB * L,), jnp.float32),
        pltpu.SemaphoreType.DMA((CPR,)),
        pltpu.SemaphoreType.DMA((NIR,)),
    ],
)


def _pre_body(drows_hbm, offs_hbm, x_hbm, w_hbm, z0_hbm,
              wbuf, didx, obuf, xb, zb):
    c = lax.axis_index("c")
    s = lax.axis_index("s")
    t = c * NS + s
    ra, rb, lo, hi = _span(offs_hbm, obuf, t)
    zeros16 = jnp.zeros((L,), jnp.float32)

    @pl.loop(0, RT)
    def _(r):
        wbuf[pl.ds(r * L, L)] = zeros16

    def flush(rcur, rcnt):
        inr = jnp.logical_and(rcur >= lo, rcur < hi)
        base = jnp.where(inr, (rcur - lo) * L, RT * L)
        wbuf[pl.ds(base, L)] = jnp.full((L,), rcnt, jnp.float32)

    def row_group(j, rc):
        rcur, rcnt = rc
        dvec = didx[pl.ds(j * L, L)]
        for l in range(L):
            dstv = dvec[l]
            changed = dstv != rcur

            @pl.when(changed)
            def _():
                flush(rcur, rcnt)

            rcnt = jnp.where(changed, 1.0, rcnt + 1.0)
            rcur = jnp.where(changed, dstv, rcur)
        return (rcur, rcnt)

    def row_body(rr, carry):
        pltpu.sync_copy(drows_hbm.at[rr], didx)
        return pl.loop(0, RW // L, init_carry=carry)(row_group)

    cur, cnt = pl.loop(ra, rb, init_carry=(jnp.int32(N), jnp.float32(0.0)))(
        row_body)
    flush(cur, cnt)

    @pl.loop(0, RT)
    def _(r):
        sl = pl.ds(r * L, L)
        wbuf[sl] = _rsqrt_vec(jnp.maximum(wbuf[sl], 1.0))
    pltpu.sync_copy(wbuf.at[pl.ds(0, RT * L)], w_hbm.at[t])

    @pl.loop(0, RT // FB)
    def _(k):
        r0 = k * FB

        @pl.when(lo + r0 < hi)
        def _():
            n0 = lo + r0
            pltpu.sync_copy(x_hbm.at[pl.ds(n0 * D, FB * D)], xb)

            @pl.loop(0, FB)
            def _(r):
                dv = wbuf[pl.ds((r0 + r) * L, L)][0]
                for q in range(D // L):
                    o = r * D + q * L
                    zb[pl.ds(o, L)] = xb[pl.ds(o, L)] * dv
            pltpu.sync_copy(zb, z0_hbm.at[pl.ds(n0 * D, FB * D)])


_pre = pl.kernel(
    _pre_body,
    out_type=(jax.ShapeDtypeStruct((NT, RT * L), jnp.float32),
              jax.ShapeDtypeStruct((N * D,), jnp.float32)),
    mesh=_mesh,
    scratch_types=[
        pltpu.VMEM((RT * L + L,), jnp.float32),
        pltpu.VMEM((RW,), jnp.int32),
        pltpu.VMEM((NT + L,), jnp.int32),
        pltpu.VMEM((FB * D,), jnp.float32),
        pltpu.VMEM((FB * D,), jnp.float32),
    ],
)


@jax.jit
def kernel(x, edge_index):
    src = edge_index[0].astype(jnp.int32)
    dst = edge_index[1].astype(jnp.int32)
    order = jnp.argsort(dst)
    src_s = jnp.pad(src[order], (0, EP - E), constant_values=0)
    dst_s = jnp.pad(dst[order], (0, EP - E), constant_values=N)
    srows = src_s.reshape(NRW, RW)
    drows = dst_s.reshape(NRW, RW)
    bounds = jnp.minimum(jnp.arange(NT + 1, dtype=jnp.int32) * RT, N)
    offs = jnp.searchsorted(dst_s, bounds, side="left").astype(jnp.int32)
    offs = jnp.pad(offs, (0, NT + L - (NT + 1)))
    xf = x.reshape(N * D)

    w, zf = _pre(drows, offs, xf)
    yf = xf
    for _ in range(PROP):
        zf, yf = _step(srows, drows, offs, w, xf, zf.reshape(N, D))
    return yf.reshape(N, D)
